# baseline (speedup 1.0000x reference)
import functools

import jax
import jax.numpy as jnp
from jax import lax
from jax.experimental import pallas as pl
from jax.experimental.pallas import tpu as pltpu
from jax.experimental.pallas import tpu_sc as plsc

N = 8192
K = 1024
D = 32
ROWS_PER_TILE = 1024
N_SC = 1024
N_TC = N - N_SC
GRID_A = N_SC // ROWS_PER_TILE
GRID_B = N_TC // ROWS_PER_TILE


def _dist_argmin(z_blk, emb_blk):
    z2 = jnp.sum(z_blk * z_blk, axis=1, keepdims=True)
    e2 = jnp.sum(emb_blk * emb_blk, axis=1)[None, :]
    prod = lax.dot_general(z_blk, emb_blk,
                           (((1,), (1,)), ((), ())),
                           preferred_element_type=jnp.float32)
    dist = z2 + e2 - 2.0 * prod
    minv = jnp.min(dist, axis=1)
    cols = lax.broadcasted_iota(jnp.int32, dist.shape, 1)
    idx = jnp.min(jnp.where(dist == minv[:, None], cols, K), axis=1)
    return idx.astype(jnp.int32), minv, cols


def _argmin_body(z_ref, emb_ref, idx_ref, minsum_ref):
    i = pl.program_id(0)
    idx, minv, _ = _dist_argmin(z_ref[...], emb_ref[...])
    idx_ref[...] = idx

    @pl.when(i == 0)
    def _init():
        minsum_ref[...] = jnp.zeros_like(minsum_ref)

    minsum_ref[...] = minsum_ref[...] + jnp.sum(minv)


def _argmin_gather_body(z_ref, emb_ref, idx_ref, minsum_ref, zq_ref):
    i = pl.program_id(0)
    emb_blk = emb_ref[...]
    idx, minv, cols = _dist_argmin(z_ref[...], emb_blk)
    idx_ref[...] = idx
    onehot = (cols == idx[:, None]).astype(jnp.float32)
    emb_hi = lax.convert_element_type(
        lax.convert_element_type(emb_blk, jnp.bfloat16), jnp.float32)
    emb_lo = emb_blk - emb_hi
    dn = (((1,), (0,)), ((), ()))
    zq_ref[...] = (
        lax.dot_general(onehot, emb_hi, dn,
                        preferred_element_type=jnp.float32)
        + lax.dot_general(onehot, emb_lo, dn,
                          preferred_element_type=jnp.float32))

    @pl.when(i == 0)
    def _init():
        minsum_ref[...] = jnp.zeros_like(minsum_ref)

    minsum_ref[...] = minsum_ref[...] + jnp.sum(minv)


def _argmin_tc_a(z, emb):
    return pl.pallas_call(
        _argmin_body,
        grid=(GRID_A,),
        in_specs=[
            pl.BlockSpec((ROWS_PER_TILE, D), lambda i: (i, 0)),
            pl.BlockSpec((K, D), lambda i: (0, 0)),
        ],
        out_specs=[
            pl.BlockSpec((ROWS_PER_TILE,), lambda i: (i,)),
            pl.BlockSpec((1, 1), lambda i: (0, 0)),
        ],
        out_shape=[
            jax.ShapeDtypeStruct((N_SC,), jnp.int32),
            jax.ShapeDtypeStruct((1, 1), jnp.float32),
        ],
    )(z, emb)


def _argmin_tc_b(z, emb):
    off = GRID_A
    return pl.pallas_call(
        _argmin_gather_body,
        grid=(GRID_B,),
        in_specs=[
            pl.BlockSpec((ROWS_PER_TILE, D), lambda i: (i + off, 0)),
            pl.BlockSpec((K, D), lambda i: (0, 0)),
        ],
        out_specs=[
            pl.BlockSpec((ROWS_PER_TILE,), lambda i: (i,)),
            pl.BlockSpec((1, 1), lambda i: (0, 0)),
            pl.BlockSpec((ROWS_PER_TILE, D), lambda i: (i, 0)),
        ],
        out_shape=[
            jax.ShapeDtypeStruct((N_TC,), jnp.int32),
            jax.ShapeDtypeStruct((1, 1), jnp.float32),
            jax.ShapeDtypeStruct((N_TC, D), jnp.float32),
        ],
    )(z, emb)


def _sc_gather(emb, idx):
    n_rows = idx.shape[0]
    info = plsc.get_sparse_core_info()
    nw = info.num_cores * info.num_subcores
    lanes = info.num_lanes
    rows_per_w = n_rows // nw
    mesh = plsc.VectorSubcoreMesh(core_axis_name="c", subcore_axis_name="s")

    @functools.partial(
        pl.kernel,
        out_type=jax.ShapeDtypeStruct((n_rows, D), jnp.float32),
        mesh=mesh,
        compiler_params=pltpu.CompilerParams(needs_layout_passes=False),
        scratch_types=[
            pltpu.VMEM((rows_per_w,), jnp.int32),
            pltpu.VMEM((K * D,), jnp.float32),
            pltpu.VMEM((rows_per_w, D), jnp.float32),
            pltpu.SemaphoreType.DMA,
        ],
    )
    def gather_kernel(emb_hbm, idx_hbm, out_hbm, idx_v, emb_v, rows_v, sem):
        wid = lax.axis_index("s") * info.num_cores + lax.axis_index("c")
        base = wid * rows_per_w
        cp = pltpu.async_copy(emb_hbm, emb_v, sem)
        pltpu.sync_copy(idx_hbm.at[pl.ds(base, rows_per_w)], idx_v)
        cp.wait()
        dcol = lax.iota(jnp.int32, lanes)

        def body(r, carry):
            rvec = jnp.full((lanes,), r, jnp.int32)
            src = (plsc.load_gather(idx_v, [rvec]) << 5) + dcol
            rows_v[r, pl.ds(0, lanes)] = plsc.load_gather(emb_v, [src])
            rows_v[r, pl.ds(lanes, lanes)] = plsc.load_gather(
                emb_v, [src + lanes])
            return carry

        lax.fori_loop(0, rows_per_w, body, 0, unroll=8)
        pltpu.sync_copy(rows_v, out_hbm.at[pl.ds(base, rows_per_w)])

    return gather_kernel(emb.reshape(-1), idx)


def kernel(z, emb):
    idx_a, ms_a = _argmin_tc_a(z, emb)
    zq_a = _sc_gather(emb, idx_a)
    idx_b, ms_b, zq_b = _argmin_tc_b(z, emb)
    z_q = jnp.concatenate([zq_a, zq_b], axis=0)
    idx = jnp.concatenate([idx_a, idx_b], axis=0)
    vq_loss = (ms_a[0, 0] + ms_b[0, 0]) * (1.25 / (N * D))
    return (z_q, idx, vq_loss)


# speedup vs baseline: 1.0684x; 1.0684x over previous
import functools

import jax
import jax.numpy as jnp
from jax import lax
from jax.experimental import pallas as pl
from jax.experimental.pallas import tpu as pltpu
from jax.experimental.pallas import tpu_sc as plsc

N = 8192
K = 1024
D = 32
ROWS_PER_TILE = 1024
N_SC = 6144
N_TC = N - N_SC
GRID_A = N_SC // ROWS_PER_TILE
GRID_B = N_TC // ROWS_PER_TILE


def _dist_argmin(z_blk, emb_blk):
    z2 = jnp.sum(z_blk * z_blk, axis=1, keepdims=True)
    e2 = jnp.sum(emb_blk * emb_blk, axis=1)[None, :]
    prod = lax.dot_general(z_blk, emb_blk,
                           (((1,), (1,)), ((), ())),
                           preferred_element_type=jnp.float32)
    dist = z2 + e2 - 2.0 * prod
    minv = jnp.min(dist, axis=1)
    cols = lax.broadcasted_iota(jnp.int32, dist.shape, 1)
    idx = jnp.min(jnp.where(dist == minv[:, None], cols, K), axis=1)
    return idx.astype(jnp.int32), minv, cols


def _argmin_body(z_ref, emb_ref, idx_ref, minsum_ref):
    i = pl.program_id(0)
    idx, minv, _ = _dist_argmin(z_ref[...], emb_ref[...])
    idx_ref[...] = idx

    @pl.when(i == 0)
    def _init():
        minsum_ref[...] = jnp.zeros_like(minsum_ref)

    minsum_ref[...] = minsum_ref[...] + jnp.sum(minv)


def _argmin_gather_body(z_ref, emb_ref, idx_ref, minsum_ref, zq_ref):
    i = pl.program_id(0)
    emb_blk = emb_ref[...]
    idx, minv, cols = _dist_argmin(z_ref[...], emb_blk)
    idx_ref[...] = idx
    onehot = (cols == idx[:, None]).astype(jnp.float32)
    emb_hi = lax.convert_element_type(
        lax.convert_element_type(emb_blk, jnp.bfloat16), jnp.float32)
    emb_lo = emb_blk - emb_hi
    dn = (((1,), (0,)), ((), ()))
    zq_ref[...] = (
        lax.dot_general(onehot, emb_hi, dn,
                        preferred_element_type=jnp.float32)
        + lax.dot_general(onehot, emb_lo, dn,
                          preferred_element_type=jnp.float32))

    @pl.when(i == 0)
    def _init():
        minsum_ref[...] = jnp.zeros_like(minsum_ref)

    minsum_ref[...] = minsum_ref[...] + jnp.sum(minv)


def _argmin_tc_a(z, emb):
    return pl.pallas_call(
        _argmin_body,
        grid=(GRID_A,),
        in_specs=[
            pl.BlockSpec((ROWS_PER_TILE, D), lambda i: (i, 0)),
            pl.BlockSpec((K, D), lambda i: (0, 0)),
        ],
        out_specs=[
            pl.BlockSpec((ROWS_PER_TILE,), lambda i: (i,)),
            pl.BlockSpec((1, 1), lambda i: (0, 0)),
        ],
        out_shape=[
            jax.ShapeDtypeStruct((N_SC,), jnp.int32),
            jax.ShapeDtypeStruct((1, 1), jnp.float32),
        ],
    )(z, emb)


def _argmin_tc_b(z, emb):
    off = GRID_A
    return pl.pallas_call(
        _argmin_gather_body,
        grid=(GRID_B,),
        in_specs=[
            pl.BlockSpec((ROWS_PER_TILE, D), lambda i: (i + off, 0)),
            pl.BlockSpec((K, D), lambda i: (0, 0)),
        ],
        out_specs=[
            pl.BlockSpec((ROWS_PER_TILE,), lambda i: (i,)),
            pl.BlockSpec((1, 1), lambda i: (0, 0)),
            pl.BlockSpec((ROWS_PER_TILE, D), lambda i: (i, 0)),
        ],
        out_shape=[
            jax.ShapeDtypeStruct((N_TC,), jnp.int32),
            jax.ShapeDtypeStruct((1, 1), jnp.float32),
            jax.ShapeDtypeStruct((N_TC, D), jnp.float32),
        ],
    )(z, emb)


def _sc_gather(emb, idx):
    n_rows = idx.shape[0]
    info = plsc.get_sparse_core_info()
    nw = info.num_cores * info.num_subcores
    lanes = info.num_lanes
    rows_per_w = n_rows // nw
    mesh = plsc.VectorSubcoreMesh(core_axis_name="c", subcore_axis_name="s")

    @functools.partial(
        pl.kernel,
        out_type=jax.ShapeDtypeStruct((n_rows, D), jnp.float32),
        mesh=mesh,
        compiler_params=pltpu.CompilerParams(needs_layout_passes=False),
        scratch_types=[
            pltpu.VMEM((rows_per_w,), jnp.int32),
            pltpu.VMEM((K * D,), jnp.float32),
            pltpu.VMEM((rows_per_w, D), jnp.float32),
            pltpu.SemaphoreType.DMA,
        ],
    )
    def gather_kernel(emb_hbm, idx_hbm, out_hbm, idx_v, emb_v, rows_v, sem):
        wid = lax.axis_index("s") * info.num_cores + lax.axis_index("c")
        base = wid * rows_per_w
        cp = pltpu.async_copy(emb_hbm, emb_v, sem)
        pltpu.sync_copy(idx_hbm.at[pl.ds(base, rows_per_w)], idx_v)
        cp.wait()
        dcol = lax.iota(jnp.int32, lanes)

        def body(r, carry):
            rvec = jnp.full((lanes,), r, jnp.int32)
            src = (plsc.load_gather(idx_v, [rvec]) << 5) + dcol
            rows_v[r, pl.ds(0, lanes)] = plsc.load_gather(emb_v, [src])
            rows_v[r, pl.ds(lanes, lanes)] = plsc.load_gather(
                emb_v, [src + lanes])
            return carry

        lax.fori_loop(0, rows_per_w, body, 0, unroll=8)
        pltpu.sync_copy(rows_v, out_hbm.at[pl.ds(base, rows_per_w)])

    return gather_kernel(emb.reshape(-1), idx)


def kernel(z, emb):
    idx_a, ms_a = _argmin_tc_a(z, emb)
    zq_a = _sc_gather(emb, idx_a)
    idx_b, ms_b, zq_b = _argmin_tc_b(z, emb)
    z_q = jnp.concatenate([zq_a, zq_b], axis=0)
    idx = jnp.concatenate([idx_a, idx_b], axis=0)
    vq_loss = (ms_a[0, 0] + ms_b[0, 0]) * (1.25 / (N * D))
    return (z_q, idx, vq_loss)


# speedup vs baseline: 1.0774x; 1.0085x over previous
import functools

import jax
import jax.numpy as jnp
from jax import lax
from jax.experimental import pallas as pl
from jax.experimental.pallas import tpu as pltpu
from jax.experimental.pallas import tpu_sc as plsc

N = 8192
K = 1024
D = 32
ROWS_PER_TILE = 1024
N_SC = 7168
N_TC = N - N_SC
GRID_A = N_SC // ROWS_PER_TILE
GRID_B = N_TC // ROWS_PER_TILE


def _dist_argmin(z_blk, emb_blk):
    z2 = jnp.sum(z_blk * z_blk, axis=1, keepdims=True)
    e2 = jnp.sum(emb_blk * emb_blk, axis=1)[None, :]
    prod = lax.dot_general(z_blk, emb_blk,
                           (((1,), (1,)), ((), ())),
                           preferred_element_type=jnp.float32)
    dist = z2 + e2 - 2.0 * prod
    minv = jnp.min(dist, axis=1)
    cols = lax.broadcasted_iota(jnp.int32, dist.shape, 1)
    idx = jnp.min(jnp.where(dist == minv[:, None], cols, K), axis=1)
    return idx.astype(jnp.int32), minv, cols


def _argmin_body(z_ref, emb_ref, idx_ref, minsum_ref):
    i = pl.program_id(0)
    idx, minv, _ = _dist_argmin(z_ref[...], emb_ref[...])
    idx_ref[...] = idx

    @pl.when(i == 0)
    def _init():
        minsum_ref[...] = jnp.zeros_like(minsum_ref)

    minsum_ref[...] = minsum_ref[...] + jnp.sum(minv)


def _argmin_gather_body(z_ref, emb_ref, idx_ref, minsum_ref, zq_ref):
    i = pl.program_id(0)
    emb_blk = emb_ref[...]
    idx, minv, cols = _dist_argmin(z_ref[...], emb_blk)
    idx_ref[...] = idx
    onehot = (cols == idx[:, None]).astype(jnp.float32)
    emb_hi = lax.convert_element_type(
        lax.convert_element_type(emb_blk, jnp.bfloat16), jnp.float32)
    emb_lo = emb_blk - emb_hi
    dn = (((1,), (0,)), ((), ()))
    zq_ref[...] = (
        lax.dot_general(onehot, emb_hi, dn,
                        preferred_element_type=jnp.float32)
        + lax.dot_general(onehot, emb_lo, dn,
                          preferred_element_type=jnp.float32))

    @pl.when(i == 0)
    def _init():
        minsum_ref[...] = jnp.zeros_like(minsum_ref)

    minsum_ref[...] = minsum_ref[...] + jnp.sum(minv)


def _argmin_tc_a(z, emb):
    return pl.pallas_call(
        _argmin_body,
        grid=(GRID_A,),
        in_specs=[
            pl.BlockSpec((ROWS_PER_TILE, D), lambda i: (i, 0)),
            pl.BlockSpec((K, D), lambda i: (0, 0)),
        ],
        out_specs=[
            pl.BlockSpec((ROWS_PER_TILE,), lambda i: (i,)),
            pl.BlockSpec((1, 1), lambda i: (0, 0)),
        ],
        out_shape=[
            jax.ShapeDtypeStruct((N_SC,), jnp.int32),
            jax.ShapeDtypeStruct((1, 1), jnp.float32),
        ],
    )(z, emb)


def _argmin_tc_b(z, emb):
    off = GRID_A
    return pl.pallas_call(
        _argmin_gather_body,
        grid=(GRID_B,),
        in_specs=[
            pl.BlockSpec((ROWS_PER_TILE, D), lambda i: (i + off, 0)),
            pl.BlockSpec((K, D), lambda i: (0, 0)),
        ],
        out_specs=[
            pl.BlockSpec((ROWS_PER_TILE,), lambda i: (i,)),
            pl.BlockSpec((1, 1), lambda i: (0, 0)),
            pl.BlockSpec((ROWS_PER_TILE, D), lambda i: (i, 0)),
        ],
        out_shape=[
            jax.ShapeDtypeStruct((N_TC,), jnp.int32),
            jax.ShapeDtypeStruct((1, 1), jnp.float32),
            jax.ShapeDtypeStruct((N_TC, D), jnp.float32),
        ],
    )(z, emb)


def _sc_gather(emb, idx):
    n_rows = idx.shape[0]
    info = plsc.get_sparse_core_info()
    nw = info.num_cores * info.num_subcores
    lanes = info.num_lanes
    rows_per_w = n_rows // nw
    mesh = plsc.VectorSubcoreMesh(core_axis_name="c", subcore_axis_name="s")

    @functools.partial(
        pl.kernel,
        out_type=jax.ShapeDtypeStruct((n_rows, D), jnp.float32),
        mesh=mesh,
        compiler_params=pltpu.CompilerParams(needs_layout_passes=False),
        scratch_types=[
            pltpu.VMEM((rows_per_w,), jnp.int32),
            pltpu.VMEM((K * D,), jnp.float32),
            pltpu.VMEM((rows_per_w, D), jnp.float32),
            pltpu.SemaphoreType.DMA,
        ],
    )
    def gather_kernel(emb_hbm, idx_hbm, out_hbm, idx_v, emb_v, rows_v, sem):
        wid = lax.axis_index("s") * info.num_cores + lax.axis_index("c")
        base = wid * rows_per_w
        cp = pltpu.async_copy(emb_hbm, emb_v, sem)
        pltpu.sync_copy(idx_hbm.at[pl.ds(base, rows_per_w)], idx_v)
        cp.wait()
        dcol = lax.iota(jnp.int32, lanes)

        def body(r, carry):
            rvec = jnp.full((lanes,), r, jnp.int32)
            src = (plsc.load_gather(idx_v, [rvec]) << 5) + dcol
            rows_v[r, pl.ds(0, lanes)] = plsc.load_gather(emb_v, [src])
            rows_v[r, pl.ds(lanes, lanes)] = plsc.load_gather(
                emb_v, [src + lanes])
            return carry

        lax.fori_loop(0, rows_per_w, body, 0, unroll=8)
        pltpu.sync_copy(rows_v, out_hbm.at[pl.ds(base, rows_per_w)])

    return gather_kernel(emb.reshape(-1), idx)


def kernel(z, emb):
    idx_a, ms_a = _argmin_tc_a(z, emb)
    zq_a = _sc_gather(emb, idx_a)
    idx_b, ms_b, zq_b = _argmin_tc_b(z, emb)
    z_q = jnp.concatenate([zq_a, zq_b], axis=0)
    idx = jnp.concatenate([idx_a, idx_b], axis=0)
    vq_loss = (ms_a[0, 0] + ms_b[0, 0]) * (1.25 / (N * D))
    return (z_q, idx, vq_loss)
